# baseline (speedup 1.0000x reference)
import functools

import jax
import jax.numpy as jnp
from jax import lax
from jax.experimental import pallas as pl
from jax.experimental.pallas import tpu as pltpu
from jax.experimental.pallas import tpu_sc as plsc

B = 16384
K = 64
NC = 2
NS = 16
NW = NC * NS
BPW = B // NW
NCHUNK = 4
CHUNK = BPW // NCHUNK
GROUPS = BPW // 16


def _cf_body(user_idx, item_idx, user_table, item_table, out,
             idx_u, idx_i, rows_u, rows_i, trans, out_v, sem):
    wid = lax.axis_index("s") * NC + lax.axis_index("c")
    base = wid * BPW

    pltpu.sync_copy(user_idx.at[wid], idx_u)
    pltpu.sync_copy(item_idx.at[wid], idx_i)

    copies = []
    for j in range(NCHUNK):
        copies.append(pltpu.async_copy(
            user_table.at[idx_u.at[j]],
            rows_u.at[pl.ds(j * CHUNK, CHUNK)], sem))
        copies.append(pltpu.async_copy(
            item_table.at[idx_i.at[j]],
            rows_i.at[pl.ds(j * CHUNK, CHUNK)], sem))
    for c in copies:
        c.wait()

    lane16 = lax.iota(jnp.int32, 16) * 16

    def group_body(g, carry):
        gbase = g * 16
        for rr in range(16):
            r = gbase + rr
            s = jnp.zeros((16,), jnp.float32)
            for q in range(K // 16):
                u = rows_u[r, pl.ds(q * 16, 16)]
                v = rows_i[r, pl.ds(q * 16, 16)]
                s = s + u * v
            trans[pl.ds(rr * 16, 16)] = s
        acc = jnp.zeros((16,), jnp.float32)
        for c in range(16):
            acc = acc + plsc.load_gather(trans, [lane16 + c])
        out_v[pl.ds(gbase, 16)] = acc
        return carry

    lax.fori_loop(0, GROUPS, group_body, 0)
    pltpu.sync_copy(out_v, out.at[pl.ds(base, BPW)])


@jax.jit
def _cf_forward(user_idx, item_idx, user_table, item_table):
    kfn = pl.kernel(
        _cf_body,
        out_type=jax.ShapeDtypeStruct((B,), jnp.float32),
        mesh=plsc.VectorSubcoreMesh(core_axis_name="c", subcore_axis_name="s"),
        compiler_params=pltpu.CompilerParams(
            needs_layout_passes=False, use_tc_tiling_on_sc=False),
        scratch_types=[
            pltpu.VMEM((NCHUNK, CHUNK), jnp.int32),
            pltpu.VMEM((NCHUNK, CHUNK), jnp.int32),
            pltpu.VMEM((BPW, K), jnp.float32),
            pltpu.VMEM((BPW, K), jnp.float32),
            pltpu.VMEM((256,), jnp.float32),
            pltpu.VMEM((BPW,), jnp.float32),
            pltpu.SemaphoreType.DMA,
        ],
    )
    return kfn(user_idx, item_idx, user_table, item_table)


def kernel(user_input, item_input, user_table, item_table):
    uidx = user_input.reshape(NW, NCHUNK, CHUNK)
    iidx = item_input.reshape(NW, NCHUNK, CHUNK)
    out = _cf_forward(uidx, iidx, user_table, item_table)
    return out.reshape(B, 1)


# speedup vs baseline: 1.9822x; 1.9822x over previous
import functools

import jax
import jax.numpy as jnp
from jax import lax
from jax.experimental import pallas as pl
from jax.experimental.pallas import tpu as pltpu
from jax.experimental.pallas import tpu_sc as plsc

B = 16384
K = 64
V = 1_000_000
NC = 2
NS = 16
NW = NC * NS
BPW = B // NW
GP = 2
NG = BPW // GP
LB = 128


def _cf_body(uidx, iidx, ut, it, out, idx_u, idx_i, ublk, iblk, stage,
             out_v, sem0, sem1):
    wid = lax.axis_index("s") * NC + lax.axis_index("c")
    base = wid * BPW

    pltpu.sync_copy(uidx.at[pl.ds(base, BPW)], idx_u)
    pltpu.sync_copy(iidx.at[pl.ds(base, BPW)], idx_i)

    sems = (sem0, sem1)
    lane = lax.iota(jnp.int32, 16)
    pvec = lane >> 3
    kof = lane & 7
    zero16 = jnp.zeros((16,), jnp.int32)

    def bcast(ref, pos):
        return plsc.load_gather(ref, [zero16 + pos])

    def scal(vec):
        return jnp.sum(vec) >> 4

    def post(g, b):
        for j in range(GP):
            cu = scal(bcast(idx_u, g * GP + j))
            ci = scal(bcast(idx_i, g * GP + j))
            qu = pl.multiple_of((cu >> 7) * LB, LB)
            qi = pl.multiple_of((ci >> 7) * LB, LB)
            pltpu.async_copy(ut.at[:, pl.ds(qu, LB)], ublk.at[b, j], sems[b])
            pltpu.async_copy(it.at[:, pl.ds(qi, LB)], iblk.at[b, j], sems[b])

    def drain(b):
        for j in range(GP):
            pltpu.make_async_copy(ut.at[:, pl.ds(0, LB)], ublk.at[b, j],
                                  sems[b]).wait()
            pltpu.make_async_copy(it.at[:, pl.ds(0, LB)], iblk.at[b, j],
                                  sems[b]).wait()

    def compute(g, b):
        u0 = bcast(idx_u, g * GP) & (LB - 1)
        u1 = bcast(idx_u, g * GP + 1) & (LB - 1)
        rvu = jnp.where(lane < 8, u0, u1)
        i0 = bcast(idx_i, g * GP) & (LB - 1)
        i1 = bcast(idx_i, g * GP + 1) & (LB - 1)
        rvi = jnp.where(lane < 8, i0, i1)
        bvec = zero16 + b
        acc = jnp.zeros((16,), jnp.float32)
        for t in range(K // 8):
            kvec = kof + (t * 8)
            uv = plsc.load_gather(ublk, [bvec, pvec, kvec, rvu])
            iv = plsc.load_gather(iblk, [bvec, pvec, kvec, rvi])
            acc = acc + uv * iv
        for sh in (1, 2, 4):
            stage[...] = acc
            acc = acc + plsc.load_gather(stage, [lane ^ sh])
        plsc.store_scatter(out_v, [g * GP + pvec], acc, mask=kof == 0)

    def loop_body(g, carry):
        post(g, 0)
        drain(0)
        compute(g, 0)
        return carry

    lax.fori_loop(0, NG, loop_body, 0)

    pltpu.sync_copy(out_v, out.at[pl.ds(base, BPW)])


@jax.jit
def _cf_forward(uidx, iidx, ut, it):
    kfn = pl.kernel(
        _cf_body,
        out_type=jax.ShapeDtypeStruct((B,), jnp.float32),
        mesh=plsc.VectorSubcoreMesh(core_axis_name="c", subcore_axis_name="s"),
        compiler_params=pltpu.CompilerParams(
            needs_layout_passes=False, use_tc_tiling_on_sc=True),
        scratch_types=[
            pltpu.VMEM((BPW,), jnp.int32),
            pltpu.VMEM((BPW,), jnp.int32),
            pltpu.VMEM((2, GP, K, LB), jnp.float32),
            pltpu.VMEM((2, GP, K, LB), jnp.float32),
            pltpu.VMEM((16,), jnp.float32),
            pltpu.VMEM((BPW,), jnp.float32),
            pltpu.SemaphoreType.DMA,
            pltpu.SemaphoreType.DMA,
        ],
    )
    return kfn(uidx, iidx, ut, it)


def kernel(user_input, item_input, user_table, item_table):
    uidx = user_input.reshape(B)
    iidx = item_input.reshape(B)
    out = _cf_forward(uidx, iidx, user_table.T, item_table.T)
    return out.reshape(B, 1)


# speedup vs baseline: 1.9931x; 1.0055x over previous
import functools

import jax
import jax.numpy as jnp
from jax import lax
from jax.experimental import pallas as pl
from jax.experimental.pallas import tpu as pltpu
from jax.experimental.pallas import tpu_sc as plsc

B = 16384
K = 64
V = 1_000_000
NC = 2
NS = 16
NW = NC * NS
BPW = B // NW
GP = 2
NG = BPW // GP
LB = 128


def _cf_body(uidx, iidx, ut, it, out, idx_u, idx_i, ublk, iblk, stage,
             out_v, sem0, sem1, sem2, sem3):
    wid = lax.axis_index("s") * NC + lax.axis_index("c")
    base = wid * BPW

    pltpu.sync_copy(uidx.at[pl.ds(base, BPW)], idx_u)
    pltpu.sync_copy(iidx.at[pl.ds(base, BPW)], idx_i)

    sems = (sem0, sem1, sem2, sem3)
    lane = lax.iota(jnp.int32, 16)
    zero16 = jnp.zeros((16,), jnp.int32)

    def bcast(ref, pos):
        return plsc.load_gather(ref, [zero16 + pos])

    def scal(vec):
        return jnp.sum(vec) >> 4

    def post(g, b):
        cu = scal(bcast(idx_u, g))
        ci = scal(bcast(idx_i, g))
        qu = pl.multiple_of((cu >> 7) * LB, LB)
        qi = pl.multiple_of((ci >> 7) * LB, LB)
        pltpu.async_copy(ut.at[:, pl.ds(qu, LB)], ublk.at[b], sems[b])
        pltpu.async_copy(it.at[:, pl.ds(qi, LB)], iblk.at[b], sems[b])

    def drain(b):
        pltpu.make_async_copy(ut.at[:, pl.ds(0, LB)], ublk.at[b],
                              sems[b]).wait()
        pltpu.make_async_copy(it.at[:, pl.ds(0, LB)], iblk.at[b],
                              sems[b]).wait()

    def compute(g, b):
        rvu = bcast(idx_u, g) & (LB - 1)
        rvi = bcast(idx_i, g) & (LB - 1)
        bvec = zero16 + b
        acc = jnp.zeros((16,), jnp.float32)
        for t in range(K // 16):
            kvec = lane + (t * 16)
            uv = plsc.load_gather(ublk, [bvec, kvec, rvu])
            iv = plsc.load_gather(iblk, [bvec, kvec, rvi])
            acc = acc + uv * iv
        for sh in (1, 2, 4, 8):
            stage[...] = acc
            acc = acc + plsc.load_gather(stage, [lane ^ sh])
        plsc.store_scatter(out_v, [zero16 + g], acc, mask=lane == 0)

    def loop_body(h, carry):
        for j in range(GP):
            post(h * GP + j, j)
        for j in range(GP):
            drain(j)
        for j in range(GP):
            compute(h * GP + j, j)
        return carry

    lax.fori_loop(0, BPW // GP, loop_body, 0)

    pltpu.sync_copy(out_v, out.at[pl.ds(base, BPW)])


@jax.jit
def _cf_forward(uidx, iidx, ut, it):
    kfn = pl.kernel(
        _cf_body,
        out_type=jax.ShapeDtypeStruct((B,), jnp.float32),
        mesh=plsc.VectorSubcoreMesh(core_axis_name="c", subcore_axis_name="s"),
        compiler_params=pltpu.CompilerParams(
            needs_layout_passes=False, use_tc_tiling_on_sc=True),
        scratch_types=[
            pltpu.VMEM((BPW,), jnp.int32),
            pltpu.VMEM((BPW,), jnp.int32),
            pltpu.VMEM((4, K, LB), jnp.float32),
            pltpu.VMEM((4, K, LB), jnp.float32),
            pltpu.VMEM((16,), jnp.float32),
            pltpu.VMEM((BPW,), jnp.float32),
            pltpu.SemaphoreType.DMA,
            pltpu.SemaphoreType.DMA,
            pltpu.SemaphoreType.DMA,
            pltpu.SemaphoreType.DMA,
        ],
    )
    return kfn(uidx, iidx, ut, it)


def kernel(user_input, item_input, user_table, item_table):
    uidx = user_input.reshape(B)
    iidx = item_input.reshape(B)
    out = _cf_forward(uidx, iidx, user_table.T, item_table.T)
    return out.reshape(B, 1)


# speedup vs baseline: 2.1197x; 1.0635x over previous
import functools

import jax
import jax.numpy as jnp
from jax import lax
from jax.experimental import pallas as pl
from jax.experimental.pallas import tpu as pltpu
from jax.experimental.pallas import tpu_sc as plsc

B = 16384
K = 64
V = 1_000_000
NC = 2
NS = 16
NW = NC * NS
BPW = B // NW
GP = 4
NG = BPW // GP
LB = 128


def _cf_body(uidx, iidx, ut, it, out, idx_u, idx_i, ublk, iblk, stage,
             out_v, sem0, sem1, sem2, sem3):
    wid = lax.axis_index("s") * NC + lax.axis_index("c")
    base = wid * BPW

    pltpu.sync_copy(uidx.at[pl.ds(base, BPW)], idx_u)
    pltpu.sync_copy(iidx.at[pl.ds(base, BPW)], idx_i)

    sems = (sem0, sem1, sem2, sem3)
    lane = lax.iota(jnp.int32, 16)
    zero16 = jnp.zeros((16,), jnp.int32)

    def bcast(ref, pos):
        return plsc.load_gather(ref, [zero16 + pos])

    def scal(vec):
        return jnp.sum(vec) >> 4

    def post(g, b):
        cu = scal(bcast(idx_u, g))
        ci = scal(bcast(idx_i, g))
        qu = pl.multiple_of((cu >> 7) * LB, LB)
        qi = pl.multiple_of((ci >> 7) * LB, LB)
        pltpu.async_copy(ut.at[:, pl.ds(qu, LB)], ublk.at[b], sems[b])
        pltpu.async_copy(it.at[:, pl.ds(qi, LB)], iblk.at[b], sems[b])

    def drain(b):
        pltpu.make_async_copy(ut.at[:, pl.ds(0, LB)], ublk.at[b],
                              sems[b]).wait()
        pltpu.make_async_copy(it.at[:, pl.ds(0, LB)], iblk.at[b],
                              sems[b]).wait()

    def compute(g, b):
        rvu = bcast(idx_u, g) & (LB - 1)
        rvi = bcast(idx_i, g) & (LB - 1)
        bvec = zero16 + b
        acc = jnp.zeros((16,), jnp.float32)
        for t in range(K // 16):
            kvec = lane + (t * 16)
            uv = plsc.load_gather(ublk, [bvec, kvec, rvu])
            iv = plsc.load_gather(iblk, [bvec, kvec, rvi])
            acc = acc + uv * iv
        for sh in (1, 2, 4, 8):
            stage[...] = acc
            acc = acc + plsc.load_gather(stage, [lane ^ sh])
        plsc.store_scatter(out_v, [zero16 + g], acc, mask=lane == 0)

    def loop_body(h, carry):
        for j in range(GP):
            post(h * GP + j, j)
        for j in range(GP):
            drain(j)
        for j in range(GP):
            compute(h * GP + j, j)
        return carry

    lax.fori_loop(0, BPW // GP, loop_body, 0)

    pltpu.sync_copy(out_v, out.at[pl.ds(base, BPW)])


@jax.jit
def _cf_forward(uidx, iidx, ut, it):
    kfn = pl.kernel(
        _cf_body,
        out_type=jax.ShapeDtypeStruct((B,), jnp.float32),
        mesh=plsc.VectorSubcoreMesh(core_axis_name="c", subcore_axis_name="s"),
        compiler_params=pltpu.CompilerParams(
            needs_layout_passes=False, use_tc_tiling_on_sc=True),
        scratch_types=[
            pltpu.VMEM((BPW,), jnp.int32),
            pltpu.VMEM((BPW,), jnp.int32),
            pltpu.VMEM((4, K, LB), jnp.float32),
            pltpu.VMEM((4, K, LB), jnp.float32),
            pltpu.VMEM((16,), jnp.float32),
            pltpu.VMEM((BPW,), jnp.float32),
            pltpu.SemaphoreType.DMA,
            pltpu.SemaphoreType.DMA,
            pltpu.SemaphoreType.DMA,
            pltpu.SemaphoreType.DMA,
        ],
    )
    return kfn(uidx, iidx, ut, it)


def kernel(user_input, item_input, user_table, item_table):
    uidx = user_input.reshape(B)
    iidx = item_input.reshape(B)
    out = _cf_forward(uidx, iidx, user_table.T, item_table.T)
    return out.reshape(B, 1)
